# baseline (speedup 1.0000x reference)
import jax
import jax.numpy as jnp
from jax import lax
from jax.experimental import pallas as pl
from jax.experimental.pallas import tpu as pltpu
from jax.experimental.pallas import tpu_sc as plsc

N = 10000
D = 128
E = 320000
NC = 2
NS = 16
NW = NC * NS
CH = 32
NCH = 320
EPW = NCH * CH
E_PAD = NW * EPW

NA = 9712
NA_ACC = NA + 8
CPT = 600
NB = N - NA
SENT = 16384

ROWS_BLK = 1000
GRID = N // ROWS_BLK


def _zero_buf(buf, rows):
    def zrow(i, carry):
        for c in range(D // 16):
            buf[i, pl.ds(c * 16, 16)] = jnp.zeros((16,), jnp.float32)
        return carry

    lax.fori_loop(0, rows, zrow, 0)


def _agg_body(x_hbm, idx, parts,
              isd, id3, rows_a, rows_b, acc2,
              is0, is1, is2, is3, sem_a, sem_b, agg_sh):
    cid = lax.axis_index("c")
    sid = lax.axis_index("s")
    wid = cid * NS + sid
    isems = (is0, is1, is2, is3)

    for r in range(3):
        for c in range(6):
            id3[r, pl.ds(c * 16, 16)] = (lax.iota(jnp.int32, 16)
                                         + (r * 96 + c * 16))

    _zero_buf(acc2, NB)
    _zero_buf(rows_a, CH)
    z0 = sid * 607
    for p in range(19):
        cnt = 32 if p < 18 else 607 - 18 * 32
        pltpu.sync_copy(rows_a.at[pl.ds(0, cnt)],
                        agg_sh.at[pl.ds(z0 + p * 32, cnt)])

    @pl.when(sid == NS - 1)
    def _():
        t0 = NS * 607
        pltpu.sync_copy(rows_a.at[pl.ds(0, NA_ACC - t0)],
                        agg_sh.at[pl.ds(t0, NA_ACC - t0)])

    plsc.subcore_barrier()

    def idx_start(j, s):
        @pl.when(j < NCH)
        def _():
            pltpu.async_copy(idx.at[wid, j],
                            isd.at[pl.ds(3 * s, 3)], isems[s])

    def idx_wait(j, s):
        @pl.when(j < NCH)
        def _():
            pltpu.make_async_copy(idx.at[wid, j],
                                  isd.at[pl.ds(3 * s, 3)], isems[s]).wait()

    def gather_start(j, s, buf, sem):
        @pl.when(j < NCH)
        def _():
            pltpu.async_copy(x_hbm.at[isd.at[3 * s]], buf, sem)

    def gather_wait(j, s, buf, sem):
        pltpu.make_async_copy(x_hbm.at[isd.at[3 * s]], buf, sem).wait()

    def scatter_add(s, buf):
        pltpu.sync_copy(buf, agg_sh.at[isd.at[3 * s + 1]], add=True)

    def scan(s, buf):
        for g in range(CH // 16):
            vv = isd[3 * s + 2, pl.ds(g * 16, 16)]
            for k in range(16):
                vk = vv[k]

                @pl.when(vk < NB)
                def _():
                    for cc in range(D // 16):
                        sl = pl.ds(cc * 16, 16)
                        acc2[vk, sl] = acc2[vk, sl] + buf[g * 16 + k, sl]

    for s in range(4):
        idx_start(s, s)
    idx_wait(0, 0)
    idx_wait(1, 1)
    gather_start(0, 0, rows_a, sem_a)
    gather_start(1, 1, rows_b, sem_b)

    def body(i, carry):
        c0 = 4 * i
        gather_wait(c0, 0, rows_a, sem_a)
        scatter_add(0, rows_a)
        scan(0, rows_a)
        idx_start(c0 + 4, 0)
        idx_wait(c0 + 2, 2)
        gather_start(c0 + 2, 2, rows_a, sem_a)
        gather_wait(c0 + 1, 1, rows_b, sem_b)
        scatter_add(1, rows_b)
        scan(1, rows_b)
        idx_start(c0 + 5, 1)
        idx_wait(c0 + 3, 3)
        gather_start(c0 + 3, 3, rows_b, sem_b)
        gather_wait(c0 + 2, 2, rows_a, sem_a)
        scatter_add(2, rows_a)
        scan(2, rows_a)
        idx_start(c0 + 6, 2)
        idx_wait(c0 + 4, 0)
        gather_start(c0 + 4, 0, rows_a, sem_a)
        gather_wait(c0 + 3, 3, rows_b, sem_b)
        scatter_add(3, rows_b)
        scan(3, rows_b)
        idx_start(c0 + 7, 3)
        idx_wait(c0 + 5, 1)
        gather_start(c0 + 5, 1, rows_b, sem_b)
        return carry

    lax.fori_loop(0, NCH // 4, body, 0)

    plsc.subcore_barrier()

    out0 = pl.multiple_of(sid * CPT, 8)
    pltpu.sync_copy(agg_sh.at[pl.ds(out0, CPT)],
                    parts.at[cid, pl.ds(out0, CPT)])

    @pl.when(sid == NS - 1)
    def _():
        tail = NS * CPT
        pltpu.sync_copy(agg_sh.at[pl.ds(tail, NA - tail)],
                        parts.at[cid, pl.ds(tail, NA - tail)])

    plsc.subcore_barrier()

    @pl.when(sid == 0)
    def _():
        _zero_buf(rows_a, CH)
        for p in range(9):
            pltpu.sync_copy(rows_a, agg_sh.at[pl.ds(p * 32, 32)])

    plsc.subcore_barrier()
    for p in range(3):
        pltpu.sync_copy(acc2.at[pl.ds(p * 96, 96)],
                        agg_sh.at[id3.at[p]], add=True)
    plsc.subcore_barrier()

    @pl.when(sid == 0)
    def _():
        pltpu.sync_copy(agg_sh.at[pl.ds(0, NB)],
                        parts.at[cid, pl.ds(NA, NB)])


def _mlp_body(pa_ref, pb_ref, x_ref, w1_ref, b1_ref, w2_ref, b2_ref, o_ref):
    t = pa_ref[0] + pb_ref[0] + x_ref[...]
    h = jnp.dot(t, w1_ref[...], preferred_element_type=jnp.float32)
    h = jnp.maximum(h + b1_ref[...], 0.0)
    o_ref[...] = (jnp.dot(h, w2_ref[...], preferred_element_type=jnp.float32)
                  + b2_ref[...])


def kernel(x, edge_index, W1, b1, W2, b2):
    src = edge_index[0].astype(jnp.int32)
    dst = edge_index[1].astype(jnp.int32)
    npad = E_PAD - E
    pad_pos = jnp.arange(npad, dtype=jnp.int32)
    src_p = jnp.concatenate([src, pad_pos & 4095])
    dst_main = jnp.concatenate([jnp.where(dst < NA, dst, NA + (dst & 7)),
                                NA + (pad_pos & 7)])
    dst_ovf = jnp.concatenate([jnp.where(dst >= NA, dst - NA, SENT),
                               jnp.full((npad,), SENT, jnp.int32)])
    idx = jnp.stack([a.reshape(NW, NCH, CH)
                     for a in (src_p, dst_main, dst_ovf)], axis=2)

    mesh = plsc.VectorSubcoreMesh(core_axis_name="c", subcore_axis_name="s")
    parts = pl.kernel(
        _agg_body,
        out_type=jax.ShapeDtypeStruct((NC, N, D), jnp.float32),
        mesh=mesh,
        scratch_types=[
            pltpu.VMEM((12, CH), jnp.int32),
            pltpu.VMEM((3, 96), jnp.int32),
            pltpu.VMEM((CH, D), jnp.float32),
            pltpu.VMEM((CH, D), jnp.float32),
            pltpu.VMEM((NB, D), jnp.float32),
            pltpu.SemaphoreType.DMA,
            pltpu.SemaphoreType.DMA,
            pltpu.SemaphoreType.DMA,
            pltpu.SemaphoreType.DMA,
            pltpu.SemaphoreType.DMA,
            pltpu.SemaphoreType.DMA,
            pltpu.VMEM_SHARED((NA_ACC, D), jnp.float32),
        ],
    )(x, idx)

    return pl.pallas_call(
        _mlp_body,
        grid=(GRID,),
        in_specs=[
            pl.BlockSpec((1, ROWS_BLK, D), lambda i: (0, i, 0)),
            pl.BlockSpec((1, ROWS_BLK, D), lambda i: (1, i, 0)),
            pl.BlockSpec((ROWS_BLK, D), lambda i: (i, 0)),
            pl.BlockSpec((D, D), lambda i: (0, 0)),
            pl.BlockSpec((1, D), lambda i: (0, 0)),
            pl.BlockSpec((D, D), lambda i: (0, 0)),
            pl.BlockSpec((1, D), lambda i: (0, 0)),
        ],
        out_specs=pl.BlockSpec((ROWS_BLK, D), lambda i: (i, 0)),
        out_shape=jax.ShapeDtypeStruct((N, D), jnp.float32),
    )(parts, parts, x, W1, b1.reshape(1, D), W2, b2.reshape(1, D))


# speedup vs baseline: 2.8983x; 2.8983x over previous
import jax
import jax.numpy as jnp
from jax import lax
from jax.experimental import pallas as pl
from jax.experimental.pallas import tpu as pltpu
from jax.experimental.pallas import tpu_sc as plsc

N = 10000
D = 128
E = 320000
NC = 2
NS = 16
NW = NC * NS
CH = 128
NCH = 80
EPW = NCH * CH
E_PAD = NW * EPW

NA = 9712
NA_ACC = NA + 8
CPT = 600
NB = N - NA
NB_ACC = NB + 8

ROWS_BLK = 1000
GRID = N // ROWS_BLK


def _zero_buf(buf, rows):
    def zrow(i, carry):
        for c in range(D // 16):
            buf[i, pl.ds(c * 16, 16)] = jnp.zeros((16,), jnp.float32)
        return carry

    lax.fori_loop(0, rows, zrow, 0)


def _agg_a_body(x_hbm, srcs, dsts, parts,
                src_v, dst_v, rows_a, rows_b, sem_a, sem_b, agg_sh):
    cid = lax.axis_index("c")
    sid = lax.axis_index("s")
    wid = cid * NS + sid

    pltpu.sync_copy(srcs.at[wid], src_v)
    pltpu.sync_copy(dsts.at[wid], dst_v)

    def remap(r, carry):
        for c in range(CH // 16):
            v = dst_v[r, pl.ds(c * 16, 16)]
            dmp = NA + (v & 7)
            dst_v[r, pl.ds(c * 16, 16)] = jnp.where(v < NA, v, dmp)
        return carry

    lax.fori_loop(0, NCH, remap, 0)

    _zero_buf(rows_a, CH)
    z0 = sid * 607
    for p in range(5):
        cnt = 128 if p < 4 else 607 - 4 * 128
        pltpu.sync_copy(rows_a.at[pl.ds(0, cnt)],
                        agg_sh.at[pl.ds(z0 + p * 128, cnt)])

    @pl.when(sid == NS - 1)
    def _():
        t0 = NS * 607
        pltpu.sync_copy(rows_a.at[pl.ds(0, NA_ACC - t0)],
                        agg_sh.at[pl.ds(t0, NA_ACC - t0)])

    plsc.subcore_barrier()

    def gather_start(j, buf, sem):
        pltpu.async_copy(x_hbm.at[src_v.at[j]], buf, sem)

    def gather_wait(j, buf, sem):
        pltpu.make_async_copy(x_hbm.at[src_v.at[j]], buf, sem).wait()

    def scatter_add(j, buf):
        pltpu.sync_copy(buf, agg_sh.at[dst_v.at[j]], add=True)

    gather_start(0, rows_a, sem_a)
    gather_start(1, rows_b, sem_b)

    def body(i, carry):
        g = 2 * i
        gather_wait(g, rows_a, sem_a)
        scatter_add(g, rows_a)

        @pl.when(g + 2 < NCH)
        def _():
            gather_start(g + 2, rows_a, sem_a)

        gather_wait(g + 1, rows_b, sem_b)
        scatter_add(g + 1, rows_b)

        @pl.when(g + 3 < NCH)
        def _():
            gather_start(g + 3, rows_b, sem_b)

        return carry

    lax.fori_loop(0, NCH // 2, body, 0)

    plsc.subcore_barrier()

    out0 = pl.multiple_of(sid * CPT, 8)
    pltpu.sync_copy(agg_sh.at[pl.ds(out0, CPT)],
                    parts.at[cid, pl.ds(out0, CPT)])

    @pl.when(sid == NS - 1)
    def _():
        tail = NS * CPT
        pltpu.sync_copy(agg_sh.at[pl.ds(tail, NA - tail)],
                        parts.at[cid, pl.ds(tail, NA - tail)])

    pltpu.sync_copy(dsts.at[wid], dst_v)

    def remap2(r, carry):
        for c in range(CH // 16):
            v = dst_v[r, pl.ds(c * 16, 16)]
            m = jnp.logical_and(v >= NA, v < N)
            dst_v[r, pl.ds(c * 16, 16)] = jnp.where(
                m, v - NA, NB + 8 + (v & 4095))
        return carry

    lax.fori_loop(0, NCH, remap2, 0)
    plsc.subcore_barrier()

    @pl.when(sid == 0)
    def _():
        _zero_buf(rows_a, CH)
        for p, cnt_p in ((0, 128), (1, 128), (2, NB_ACC - 256)):
            pltpu.sync_copy(rows_a.at[pl.ds(0, cnt_p)],
                            agg_sh.at[pl.ds(p * 128, cnt_p)])

    plsc.subcore_barrier()

    gather_start(0, rows_a, sem_a)
    gather_start(1, rows_b, sem_b)
    lax.fori_loop(0, NCH // 2, body, 0)

    plsc.subcore_barrier()

    @pl.when(sid == 0)
    def _():
        pltpu.sync_copy(agg_sh.at[pl.ds(0, NB)],
                        parts.at[cid, pl.ds(NA, NB)])


def _mlp_body(pa_ref, pb_ref, x_ref, w1_ref, b1_ref, w2_ref, b2_ref, o_ref):
    t = pa_ref[0] + pb_ref[0] + x_ref[...]
    h = jnp.dot(t, w1_ref[...], preferred_element_type=jnp.float32)
    h = jnp.maximum(h + b1_ref[...], 0.0)
    o_ref[...] = (jnp.dot(h, w2_ref[...], preferred_element_type=jnp.float32)
                  + b2_ref[...])


def kernel(x, edge_index, W1, b1, W2, b2):
    src = edge_index[0].astype(jnp.int32)
    dst = edge_index[1].astype(jnp.int32)
    npad = E_PAD - E
    pad_pos = jnp.arange(npad, dtype=jnp.int32)
    src_p = jnp.concatenate([src, pad_pos & 4095]).reshape(NW, NCH, CH)
    dst_p = jnp.concatenate([dst, jnp.full((npad,), N, jnp.int32)]
                            ).reshape(NW, NCH, CH)

    mesh = plsc.VectorSubcoreMesh(core_axis_name="c", subcore_axis_name="s")
    parts = pl.kernel(
        _agg_a_body,
        out_type=jax.ShapeDtypeStruct((NC, N, D), jnp.float32),
        mesh=mesh,
        scratch_types=[
            pltpu.VMEM((NCH, CH), jnp.int32),
            pltpu.VMEM((NCH, CH), jnp.int32),
            pltpu.VMEM((CH, D), jnp.float32),
            pltpu.VMEM((CH, D), jnp.float32),
            pltpu.SemaphoreType.DMA,
            pltpu.SemaphoreType.DMA,
            pltpu.VMEM_SHARED((NA_ACC, D), jnp.float32),
        ],
    )(x, src_p, dst_p)

    return pl.pallas_call(
        _mlp_body,
        grid=(GRID,),
        in_specs=[
            pl.BlockSpec((1, ROWS_BLK, D), lambda i: (0, i, 0)),
            pl.BlockSpec((1, ROWS_BLK, D), lambda i: (1, i, 0)),
            pl.BlockSpec((ROWS_BLK, D), lambda i: (i, 0)),
            pl.BlockSpec((D, D), lambda i: (0, 0)),
            pl.BlockSpec((1, D), lambda i: (0, 0)),
            pl.BlockSpec((D, D), lambda i: (0, 0)),
            pl.BlockSpec((1, D), lambda i: (0, 0)),
        ],
        out_specs=pl.BlockSpec((ROWS_BLK, D), lambda i: (i, 0)),
        out_shape=jax.ShapeDtypeStruct((N, D), jnp.float32),
    )(parts, parts, x, W1, b1.reshape(1, D), W2, b2.reshape(1, D))
